# baseline (speedup 1.0000x reference)
import functools
import math

import jax
import jax.numpy as jnp
from jax.experimental import pallas as pl
from jax.experimental.pallas import tpu as pltpu

_B = 128
_V = 100000
_CHUNK = 4096
_NCHUNKS = (_V + _CHUNK - 1) // _CHUNK

_BETA = 0.7


def _lse_kernel(l1_ref, l2_ref, val_ref, out_ref, m1, s1, g1, m2, s2, g2):
    pid = pl.program_id(0)

    @pl.when(pid == 0)
    def _init():
        neg_inf = jnp.full((_B, 1), -jnp.inf, jnp.float32)
        zero = jnp.zeros((_B, 1), jnp.float32)
        m1[...] = neg_inf
        m2[...] = neg_inf
        s1[...] = zero
        s2[...] = zero
        g1[...] = zero
        g2[...] = zero

    cols = pid * _CHUNK + jax.lax.broadcasted_iota(jnp.int32, (_B, _CHUNK), 1)
    valid = cols < _V
    eq = cols == val_ref[...]

    def _update(x_raw, m_ref, s_ref, g_ref):
        x = jnp.where(valid, x_raw, -jnp.inf)
        m_old = m_ref[...]
        m_new = jnp.maximum(m_old, jnp.max(x, axis=1, keepdims=True))
        s_ref[...] = s_ref[...] * jnp.exp(m_old - m_new) + jnp.sum(
            jnp.exp(x - m_new), axis=1, keepdims=True
        )
        m_ref[...] = m_new
        g_ref[...] += jnp.sum(jnp.where(eq, x_raw, 0.0), axis=1, keepdims=True)

    _update(l1_ref[...], m1, s1, g1)
    _update(l2_ref[...], m2, s2, g2)

    @pl.when(pid == _NCHUNKS - 1)
    def _finish():
        lp1 = g1[...] - m1[...] - jnp.log(s1[...]) + math.log(_BETA)
        lp2 = g2[...] - m2[...] - jnp.log(s2[...]) + math.log(1.0 - _BETA)
        mx = jnp.maximum(lp1, lp2)
        out_ref[...] = mx + jnp.log(jnp.exp(lp1 - mx) + jnp.exp(lp2 - mx))


@jax.jit
def kernel(logits_1, logits_2, value):
    val2d = value.astype(jnp.int32).reshape(_B, 1)
    out = pl.pallas_call(
        _lse_kernel,
        grid=(_NCHUNKS,),
        in_specs=[
            pl.BlockSpec((_B, _CHUNK), lambda i: (0, i)),
            pl.BlockSpec((_B, _CHUNK), lambda i: (0, i)),
            pl.BlockSpec((_B, 1), lambda i: (0, 0)),
        ],
        out_specs=pl.BlockSpec((_B, 1), lambda i: (0, 0)),
        out_shape=jax.ShapeDtypeStruct((_B, 1), jnp.float32),
        scratch_shapes=[pltpu.VMEM((_B, 1), jnp.float32) for _ in range(6)],
    )(logits_1, logits_2, val2d)
    return out[:, 0][None, :]


# speedup vs baseline: 2.6616x; 2.6616x over previous
import math

import jax
import jax.numpy as jnp
from jax.experimental import pallas as pl
from jax.experimental.pallas import tpu as pltpu

_B = 128
_V = 100000
_CHUNK = 4000
_NCHUNKS = _V // _CHUNK

_BETA = 0.7


def _lse_kernel(l1_ref, l2_ref, val_ref, out_ref, m1, s1, g1, m2, s2, g2):
    pid = pl.program_id(0)

    @pl.when(pid == 0)
    def _init():
        neg_inf = jnp.full((1, _B), -jnp.inf, jnp.float32)
        zero = jnp.zeros((1, _B), jnp.float32)
        m1[...] = neg_inf
        m2[...] = neg_inf
        s1[...] = zero
        s2[...] = zero
        g1[...] = zero
        g2[...] = zero

    rows = pid * _CHUNK + jax.lax.broadcasted_iota(jnp.int32, (_CHUNK, _B), 0)
    eq = rows == val_ref[...]

    def _update(x, m_ref, s_ref, g_ref):
        m_old = m_ref[...]
        m_new = jnp.maximum(m_old, jnp.max(x, axis=0, keepdims=True))
        s_ref[...] = s_ref[...] * jnp.exp(m_old - m_new) + jnp.sum(
            jnp.exp(x - m_new), axis=0, keepdims=True
        )
        m_ref[...] = m_new
        g_ref[...] += jnp.sum(jnp.where(eq, x, 0.0), axis=0, keepdims=True)

    _update(l1_ref[...], m1, s1, g1)
    _update(l2_ref[...], m2, s2, g2)

    @pl.when(pid == _NCHUNKS - 1)
    def _finish():
        lp1 = g1[...] - m1[...] - jnp.log(s1[...]) + math.log(_BETA)
        lp2 = g2[...] - m2[...] - jnp.log(s2[...]) + math.log(1.0 - _BETA)
        mx = jnp.maximum(lp1, lp2)
        out_ref[...] = mx + jnp.log(jnp.exp(lp1 - mx) + jnp.exp(lp2 - mx))


@jax.jit
def kernel(logits_1, logits_2, value):
    lt1 = logits_1.T
    lt2 = logits_2.T
    val2d = value.astype(jnp.int32).reshape(1, _B)
    return pl.pallas_call(
        _lse_kernel,
        grid=(_NCHUNKS,),
        in_specs=[
            pl.BlockSpec((_CHUNK, _B), lambda i: (i, 0)),
            pl.BlockSpec((_CHUNK, _B), lambda i: (i, 0)),
            pl.BlockSpec((1, _B), lambda i: (0, 0)),
        ],
        out_specs=pl.BlockSpec((1, _B), lambda i: (0, 0)),
        out_shape=jax.ShapeDtypeStruct((1, _B), jnp.float32),
        scratch_shapes=[pltpu.VMEM((1, _B), jnp.float32) for _ in range(6)],
    )(lt1, lt2, val2d)
